# baseline (speedup 1.0000x reference)
import functools

import jax
import jax.numpy as jnp
from jax.experimental import pallas as pl
from jax.experimental.pallas import tpu as pltpu

_B, _N, _C = 8, 576, 384
_K = _N // 4
_BN = _B * _N
_ITERS = 10

_DE = dict(precision=jax.lax.Precision.DEFAULT,
           preferred_element_type=jnp.float32)
_HI = dict(precision=jax.lax.Precision.HIGHEST,
           preferred_element_type=jnp.float32)

_VMEM_PARAMS = pltpu.CompilerParams(vmem_limit_bytes=100 * 1024 * 1024)
_VSPEC = pl.BlockSpec(memory_space=pltpu.VMEM)


def _assign(c, xT, a2t, b2):
    xct = jax.lax.dot_general(c, xT, (((1,), (0,)), ((), ())), **_DE)
    dd = jnp.sqrt(jnp.maximum(a2t + b2 - 2.0 * xct, 0.0))
    mn = jnp.min(dd, axis=0, keepdims=True)
    ksub = jax.lax.broadcasted_iota(jnp.int32, (_K, _BN), 0)
    ci = jnp.min(jnp.where(dd == mn, ksub, 2 ** 30), axis=0, keepdims=True)
    return (ksub == ci).astype(jnp.float32)


def _seg_mean(ohT, pts):
    sums = jax.lax.dot_general(ohT, pts, (((1,), (0,)), ((), ())), **_HI)
    counts = jnp.sum(ohT, axis=1, keepdims=True)
    return jnp.where(counts > 0, sums / jnp.maximum(counts, 1.0),
                     jnp.zeros_like(sums))


def _iter_body(x_ref, xT_ref, a2t_ref, b2_ref, c_ref, cnext_ref):
    ohT = _assign(c_ref[...], xT_ref[...], a2t_ref[...], b2_ref[...])
    cnext_ref[...] = _seg_mean(ohT, x_ref[...])


def _final_body(x_ref, xT_ref, a2t_ref, b2_ref, c_ref, out_ref):
    x = x_ref[...]
    ohT = _assign(c_ref[...], xT_ref[...], a2t_ref[...], b2_ref[...])
    for b in range(_B):
        lo, hi = b * _N, (b + 1) * _N
        out_ref[b, :, :] = _seg_mean(ohT[:, lo:hi], x[lo:hi])


_iter_call = pl.pallas_call(
    _iter_body,
    out_shape=jax.ShapeDtypeStruct((_K, _C), jnp.float32),
    in_specs=[_VSPEC] * 5,
    compiler_params=_VMEM_PARAMS)

_final_call = pl.pallas_call(
    _final_body,
    out_shape=jax.ShapeDtypeStruct((_B, _K, _C), jnp.float32),
    in_specs=[_VSPEC] * 5,
    compiler_params=_VMEM_PARAMS)


@functools.partial(jax.jit, static_argnames=())
def kernel(x, perm):
    x_flat = x.reshape(_BN, _C)
    xT = x_flat.T
    a2t = jnp.sum(x_flat * x_flat, axis=1, keepdims=True).T
    c = x_flat[perm[:_K]]
    for _ in range(_ITERS - 1):
        b2 = jnp.sum(c * c, axis=1, keepdims=True)
        c = _iter_call(x_flat, xT, a2t, b2, c)
    b2 = jnp.sum(c * c, axis=1, keepdims=True)
    return _final_call(x_flat, xT, a2t, b2, c)
